# Initial kernel scaffold; baseline (speedup 1.0000x reference)
import jax
import jax.numpy as jnp
from jax.experimental import pallas as pl


def kernel(prefix, embedding):
    raise NotImplementedError("write your pallas kernel here")


# speedup vs baseline: 1.6236x; 1.6236x over previous
import functools

import jax
import jax.numpy as jnp
from jax import lax
from jax.experimental import pallas as pl
from jax.experimental.pallas import tpu as pltpu
from jax.experimental.pallas import tpu_sc as plsc

PRE_SEQ_LEN = 128
HIDDEN = 2048
NUM_LAYERS = 24
ROW_DIM = NUM_LAYERS * 2 * HIDDEN
BATCH = 4
B_FLAT = BATCH * PRE_SEQ_LEN

NUM_CORES = 2
NUM_SUBCORES = 16
NUM_WORKERS = NUM_CORES * NUM_SUBCORES
B_PER_W = B_FLAT // NUM_WORKERS

NUM_CHUNKS = 16
CHUNK = ROW_DIM // NUM_CHUNKS


def _make_sc_gather(interpret=False):
    mesh = plsc.VectorSubcoreMesh(core_axis_name="c", subcore_axis_name="s")

    @functools.partial(
        pl.kernel,
        out_type=jax.ShapeDtypeStruct((B_FLAT, ROW_DIM), jnp.float32),
        mesh=mesh,
        scratch_types=[
            pltpu.VMEM((B_PER_W,), jnp.int32),
            pltpu.VMEM((B_PER_W, CHUNK), jnp.float32),
            pltpu.SemaphoreType.DMA,
        ],
        interpret=interpret,
    )
    def sc_gather(idx_hbm, table_hbm, out_hbm, idx_v, buf, sem):
        wid = lax.axis_index("s") * NUM_CORES + lax.axis_index("c")
        base = wid * B_PER_W
        pltpu.sync_copy(idx_hbm.at[pl.ds(base, B_PER_W)], idx_v)

        def body(c, _):
            col = c * CHUNK
            pltpu.async_copy(
                table_hbm.at[idx_v, pl.ds(col, CHUNK)], buf, sem
            ).wait()
            pltpu.sync_copy(buf, out_hbm.at[pl.ds(base, B_PER_W), pl.ds(col, CHUNK)])
            return _

        lax.fori_loop(0, NUM_CHUNKS, body, 0, unroll=False)

    return sc_gather


def kernel(prefix, embedding):
    idx = prefix.reshape(B_FLAT)
    out = _make_sc_gather()(idx, embedding)
    return out.reshape(BATCH, PRE_SEQ_LEN, ROW_DIM)


# speedup vs baseline: 1.7016x; 1.0480x over previous
import functools

import jax
import jax.numpy as jnp
from jax import lax
from jax.experimental import pallas as pl
from jax.experimental.pallas import tpu as pltpu
from jax.experimental.pallas import tpu_sc as plsc

PRE_SEQ_LEN = 128
HIDDEN = 2048
NUM_LAYERS = 24
ROW_DIM = NUM_LAYERS * 2 * HIDDEN
BATCH = 4
B_FLAT = BATCH * PRE_SEQ_LEN

NUM_CORES = 2
NUM_SUBCORES = 16
NUM_WORKERS = NUM_CORES * NUM_SUBCORES
B_PER_W = B_FLAT // NUM_WORKERS

NUM_CHUNKS = 32
CHUNK = ROW_DIM // NUM_CHUNKS
NUM_PAIRS = NUM_CHUNKS // 2


def _make_sc_gather(interpret=False):
    mesh = plsc.VectorSubcoreMesh(core_axis_name="c", subcore_axis_name="s")

    @functools.partial(
        pl.kernel,
        out_type=jax.ShapeDtypeStruct((B_FLAT, ROW_DIM), jnp.float32),
        mesh=mesh,
        scratch_types=[
            pltpu.VMEM((B_PER_W,), jnp.int32),
            pltpu.VMEM((B_PER_W, CHUNK), jnp.float32),
            pltpu.VMEM((B_PER_W, CHUNK), jnp.float32),
            pltpu.SemaphoreType.DMA,
            pltpu.SemaphoreType.DMA,
            pltpu.SemaphoreType.DMA,
            pltpu.SemaphoreType.DMA,
        ],
        interpret=interpret,
    )
    def sc_gather(idx_hbm, table_hbm, out_hbm, idx_v, buf0, buf1, rd0, rd1, wr0, wr1):
        wid = lax.axis_index("s") * NUM_CORES + lax.axis_index("c")
        base = wid * B_PER_W
        pltpu.sync_copy(idx_hbm.at[pl.ds(base, B_PER_W)], idx_v)

        def gth(c, buf, sem):
            return pltpu.make_async_copy(
                table_hbm.at[idx_v, pl.ds(c * CHUNK, CHUNK)], buf, sem
            )

        def put(c, buf, sem):
            return pltpu.make_async_copy(
                buf, out_hbm.at[pl.ds(base, B_PER_W), pl.ds(c * CHUNK, CHUNK)], sem
            )

        gth(0, buf0, rd0).start()

        def body(j, _):
            c0 = 2 * j
            c1 = c0 + 1
            gth(c0, buf0, rd0).wait()

            @pl.when(j > 0)
            def _wait_prev_wr1():
                put(c1 - 2, buf1, wr1).wait()

            gth(c1, buf1, rd1).start()
            put(c0, buf0, wr0).start()
            gth(c1, buf1, rd1).wait()
            put(c0, buf0, wr0).wait()
            put(c1, buf1, wr1).start()

            @pl.when(j + 1 < NUM_PAIRS)
            def _next_gather():
                gth(c0 + 2, buf0, rd0).start()

            return _

        lax.fori_loop(0, NUM_PAIRS, body, 0, unroll=False)
        put(NUM_CHUNKS - 1, buf1, wr1).wait()

    return sc_gather


def kernel(prefix, embedding):
    idx = prefix.reshape(B_FLAT)
    out = _make_sc_gather()(idx, embedding)
    return out.reshape(BATCH, PRE_SEQ_LEN, ROW_DIM)


# speedup vs baseline: 2.0504x; 1.2050x over previous
import functools

import jax
import jax.numpy as jnp
from jax import lax
from jax.experimental import pallas as pl
from jax.experimental.pallas import tpu as pltpu
from jax.experimental.pallas import tpu_sc as plsc

PRE_SEQ_LEN = 128
HIDDEN = 2048
NUM_LAYERS = 24
ROW_DIM = NUM_LAYERS * 2 * HIDDEN
BATCH = 4
B_FLAT = BATCH * PRE_SEQ_LEN

NUM_CORES = 2
NUM_SUBCORES = 16
ROWS_PER_TEC = B_FLAT // NUM_SUBCORES
HALF_COLS = ROW_DIM // NUM_CORES
CHUNK = 6144
NUM_CHUNKS = HALF_COLS // CHUNK
STAGE_ROWS = PRE_SEQ_LEN // NUM_SUBCORES


def _make_sc_gather(interpret=False):
    mesh = plsc.VectorSubcoreMesh(core_axis_name="c", subcore_axis_name="s")

    @functools.partial(
        pl.kernel,
        out_type=jax.ShapeDtypeStruct((B_FLAT, ROW_DIM), jnp.float32),
        mesh=mesh,
        scratch_types=[
            pltpu.VMEM((ROWS_PER_TEC,), jnp.int32),
            pltpu.VMEM_SHARED((PRE_SEQ_LEN, CHUNK), jnp.float32),
            pltpu.VMEM_SHARED((PRE_SEQ_LEN, CHUNK), jnp.float32),
            pltpu.SemaphoreType.DMA,
            pltpu.SemaphoreType.DMA,
            pltpu.SemaphoreType.DMA,
        ],
        interpret=interpret,
    )
    def sc_gather(idx_hbm, table_hbm, out_hbm, idx_v, sp0, sp1, st_sem, wr0, wr1):
        sc = lax.axis_index("c")
        tec = lax.axis_index("s")
        row0 = tec * ROWS_PER_TEC
        col0 = sc * HALF_COLS
        srow = tec * STAGE_ROWS
        pltpu.sync_copy(idx_hbm.at[pl.ds(row0, ROWS_PER_TEC)], idx_v)
        idx_lo = idx_v[pl.ds(0, 16)]
        idx_hi = idx_v[pl.ds(16, 16)]
        idx_s = [idx_lo[i] for i in range(16)] + [idx_hi[i] for i in range(16)]

        def stage(c, sp):
            return pltpu.make_async_copy(
                table_hbm.at[pl.ds(srow, STAGE_ROWS), pl.ds(col0 + c * CHUNK, CHUNK)],
                sp.at[pl.ds(srow, STAGE_ROWS)],
                st_sem,
            )

        def row_write(c, sp, i, wsem):
            return pltpu.make_async_copy(
                sp.at[pl.ds(idx_s[i], 1)],
                out_hbm.at[pl.ds(row0 + i, 1), pl.ds(col0 + c * CHUNK, CHUNK)],
                wsem,
            )

        def step(c, sp, sp_other, wsem, wsem_other):
            stage(c, sp).wait()

            @pl.when(c >= 1)
            def _drain_prev_writes():
                for i in range(ROWS_PER_TEC):
                    row_write(c - 1, sp_other, i, wsem_other).wait()

            plsc.subcore_barrier()

            @pl.when(c + 1 < NUM_CHUNKS)
            def _next_stage():
                stage(c + 1, sp_other).start()

            for i in range(ROWS_PER_TEC):
                row_write(c, sp, i, wsem).start()

        stage(0, sp0).start()

        def body(j, _):
            step(2 * j, sp0, sp1, wr0, wr1)
            step(2 * j + 1, sp1, sp0, wr1, wr0)
            return _

        lax.fori_loop(0, NUM_CHUNKS // 2, body, 0, unroll=False)
        for i in range(ROWS_PER_TEC):
            row_write(NUM_CHUNKS - 1, sp1, i, wr1).wait()

    return sc_gather


def kernel(prefix, embedding):
    idx = prefix.reshape(B_FLAT)
    out = _make_sc_gather()(idx, embedding)
    return out.reshape(BATCH, PRE_SEQ_LEN, ROW_DIM)


# speedup vs baseline: 2.2919x; 1.1178x over previous
import functools

import jax
import jax.numpy as jnp
from jax import lax
from jax.experimental import pallas as pl
from jax.experimental.pallas import tpu as pltpu
from jax.experimental.pallas import tpu_sc as plsc

PRE_SEQ_LEN = 128
ROW_DIM = 98304
BATCH = 4
B_FLAT = 512

NUM_CORES = 2
NUM_SUBCORES = 16
ROWS_PER_TEC = B_FLAT // NUM_SUBCORES
HALF_COLS = ROW_DIM // NUM_CORES
CHUNK = 1024
NUM_CHUNKS = HALF_COLS // CHUNK
STAGE_ROWS = PRE_SEQ_LEN // NUM_SUBCORES


def make_sc_gather():
    mesh = plsc.VectorSubcoreMesh(core_axis_name="c", subcore_axis_name="s")

    @functools.partial(
        pl.kernel,
        out_type=jax.ShapeDtypeStruct((B_FLAT, ROW_DIM), jnp.float32),
        mesh=mesh,
        scratch_types=[
            pltpu.VMEM((ROWS_PER_TEC,), jnp.int32),
            pltpu.VMEM((ROWS_PER_TEC, CHUNK), jnp.float32),
            pltpu.VMEM((ROWS_PER_TEC, CHUNK), jnp.float32),
            pltpu.VMEM_SHARED((PRE_SEQ_LEN, CHUNK), jnp.float32),
            pltpu.VMEM_SHARED((PRE_SEQ_LEN, CHUNK), jnp.float32),
            pltpu.SemaphoreType.DMA,
            pltpu.SemaphoreType.DMA,
            pltpu.SemaphoreType.DMA,
            pltpu.SemaphoreType.DMA,
        ],
    )
    def sc_gather(
        idx_hbm, table_hbm, out_hbm,
        idx_v, tb0, tb1, sp0, sp1, st_sem, xb_sem, wr0, wr1,
    ):
        sc = lax.axis_index("c")
        tec = lax.axis_index("s")
        row0 = tec * ROWS_PER_TEC
        col0 = sc * HALF_COLS
        srow = tec * STAGE_ROWS
        pltpu.sync_copy(idx_hbm.at[pl.ds(row0, ROWS_PER_TEC)], idx_v)
        idx_lo = idx_v[pl.ds(0, 16)]
        idx_hi = idx_v[pl.ds(16, 16)]
        idx_s = [idx_lo[i] for i in range(16)] + [idx_hi[i] for i in range(16)]

        def stage(c, sp):
            return pltpu.make_async_copy(
                table_hbm.at[pl.ds(srow, STAGE_ROWS), pl.ds(col0 + c * CHUNK, CHUNK)],
                sp.at[pl.ds(srow, STAGE_ROWS)],
                st_sem,
            )

        def xbar(sp, tb, i):
            return pltpu.make_async_copy(
                sp.at[pl.ds(idx_s[i], 1)], tb.at[pl.ds(i, 1)], xb_sem
            )

        def write(c, tb, wsem):
            return pltpu.make_async_copy(
                tb,
                out_hbm.at[pl.ds(row0, ROWS_PER_TEC), pl.ds(col0 + c * CHUNK, CHUNK)],
                wsem,
            )

        def step(c, sp, sp_other, tb, wsem):
            stage(c, sp).wait()
            plsc.subcore_barrier()

            @pl.when(c + 1 < NUM_CHUNKS)
            def _next_stage():
                stage(c + 1, sp_other).start()

            @pl.when(c >= 2)
            def _wait_old_write():
                write(c - 2, tb, wsem).wait()

            for i in range(ROWS_PER_TEC):
                xbar(sp, tb, i).start()
            for i in range(ROWS_PER_TEC):
                xbar(sp, tb, i).wait()
            write(c, tb, wsem).start()

        stage(0, sp0).start()

        def body(j, _):
            step(2 * j, sp0, sp1, tb0, wr0)
            step(2 * j + 1, sp1, sp0, tb1, wr1)
            return _

        lax.fori_loop(0, NUM_CHUNKS // 2, body, 0, unroll=False)
        write(NUM_CHUNKS - 2, tb0, wr0).wait()
        write(NUM_CHUNKS - 1, tb1, wr1).wait()

    return sc_gather


def kernel(prefix, embedding):
    idx = prefix.reshape(B_FLAT)
    out = make_sc_gather()(idx, embedding)
    return out.reshape(BATCH, PRE_SEQ_LEN, ROW_DIM)


# speedup vs baseline: 2.4777x; 1.0811x over previous
import functools

import jax
import jax.numpy as jnp
from jax import lax
from jax.experimental import pallas as pl
from jax.experimental.pallas import tpu as pltpu
from jax.experimental.pallas import tpu_sc as plsc

PRE_SEQ_LEN = 128
ROW_DIM = 98304
BATCH = 4
B_FLAT = 512

NUM_CORES = 2
NUM_SUBCORES = 16
ROWS_PER_TEC = B_FLAT // NUM_SUBCORES
A_ROWS = 16
B_ROWS = ROWS_PER_TEC - A_ROWS
HALF_COLS = ROW_DIM // NUM_CORES
CHUNK = 4096
NUM_CHUNKS = HALF_COLS // CHUNK
SUB = 1024
NUM_SUBS = CHUNK // SUB
STAGE_ROWS = PRE_SEQ_LEN // NUM_SUBCORES


def make_sc_gather():
    mesh = plsc.VectorSubcoreMesh(core_axis_name="c", subcore_axis_name="s")

    @functools.partial(
        pl.kernel,
        out_type=jax.ShapeDtypeStruct((B_FLAT, ROW_DIM), jnp.float32),
        mesh=mesh,
        scratch_types=[
            pltpu.VMEM((ROWS_PER_TEC,), jnp.int32),
            pltpu.VMEM((B_ROWS, SUB), jnp.float32),
            pltpu.VMEM((B_ROWS, SUB), jnp.float32),
            pltpu.VMEM_SHARED((PRE_SEQ_LEN, CHUNK), jnp.float32),
            pltpu.VMEM_SHARED((PRE_SEQ_LEN, CHUNK), jnp.float32),
            pltpu.SemaphoreType.DMA,
            pltpu.SemaphoreType.DMA,
            pltpu.SemaphoreType.DMA,
            pltpu.SemaphoreType.DMA,
            pltpu.SemaphoreType.DMA,
            pltpu.SemaphoreType.DMA,
        ],
    )
    def sc_gather(
        idx_hbm, table_hbm, out_hbm,
        idx_v, tb0, tb1, sp0, sp1, st_sem, xb_sem, aw0, aw1, bw0, bw1,
    ):
        sc = lax.axis_index("c")
        tec = lax.axis_index("s")
        row0 = tec * ROWS_PER_TEC
        col0 = sc * HALF_COLS
        srow = tec * STAGE_ROWS
        pltpu.sync_copy(idx_hbm.at[pl.ds(row0, ROWS_PER_TEC)], idx_v)
        idx_lo = idx_v[pl.ds(0, 16)]
        idx_hi = idx_v[pl.ds(16, 16)]
        idx_s = [idx_lo[i] for i in range(16)] + [idx_hi[i] for i in range(16)]

        def stage(c, sp):
            return pltpu.make_async_copy(
                table_hbm.at[pl.ds(srow, STAGE_ROWS), pl.ds(col0 + c * CHUNK, CHUNK)],
                sp.at[pl.ds(srow, STAGE_ROWS)],
                st_sem,
            )

        def a_write(c, sp, i, sem):
            return pltpu.make_async_copy(
                sp.at[pl.ds(idx_s[i], 1)],
                out_hbm.at[pl.ds(row0 + i, 1), pl.ds(col0 + c * CHUNK, CHUNK)],
                sem,
            )

        def b_fill(sp, s, tb, i):
            return pltpu.make_async_copy(
                sp.at[pl.ds(idx_s[A_ROWS + i], 1), pl.ds(s * SUB, SUB)],
                tb.at[pl.ds(i, 1)],
                xb_sem,
            )

        def b_write(c, s, tb, sem):
            return pltpu.make_async_copy(
                tb,
                out_hbm.at[
                    pl.ds(row0 + A_ROWS, B_ROWS),
                    pl.ds(col0 + c * CHUNK + s * SUB, SUB),
                ],
                sem,
            )

        def step(c, sp, sp_other, asem, asem_other):
            stage(c, sp).wait()

            @pl.when(c >= 1)
            def _drain_a_prev():
                for i in range(A_ROWS):
                    a_write(c - 1, sp_other, i, asem_other).wait()

            plsc.subcore_barrier()

            @pl.when(c + 1 < NUM_CHUNKS)
            def _next_stage():
                stage(c + 1, sp_other).start()

            for i in range(A_ROWS):
                a_write(c, sp, i, asem).start()

            for s in range(NUM_SUBS):
                tb = tb0 if s % 2 == 0 else tb1
                bsem = bw0 if s % 2 == 0 else bw1

                if s >= 2:
                    b_write(c, s - 2, tb, bsem).wait()
                else:

                    @pl.when(c > 0)
                    def _drain_b_old(c=c, s=s, tb=tb, bsem=bsem):
                        b_write(c - 1, s + 2, tb, bsem).wait()

                for i in range(B_ROWS):
                    b_fill(sp, s, tb, i).start()
                for i in range(B_ROWS):
                    b_fill(sp, s, tb, i).wait()
                b_write(c, s, tb, bsem).start()

        stage(0, sp0).start()

        def body(j, _):
            step(2 * j, sp0, sp1, aw0, aw1)
            step(2 * j + 1, sp1, sp0, aw1, aw0)
            return _

        lax.fori_loop(0, NUM_CHUNKS // 2, body, 0, unroll=False)
        for i in range(A_ROWS):
            a_write(NUM_CHUNKS - 1, sp1, i, aw1).wait()
        b_write(NUM_CHUNKS - 1, NUM_SUBS - 2, tb0, bw0).wait()
        b_write(NUM_CHUNKS - 1, NUM_SUBS - 1, tb1, bw1).wait()

    return sc_gather


def kernel(prefix, embedding):
    idx = prefix.reshape(B_FLAT)
    out = make_sc_gather()(idx, embedding)
    return out.reshape(BATCH, PRE_SEQ_LEN, ROW_DIM)


# speedup vs baseline: 2.5067x; 1.0117x over previous
import functools

import jax
import jax.numpy as jnp
from jax import lax
from jax.experimental import pallas as pl
from jax.experimental.pallas import tpu as pltpu
from jax.experimental.pallas import tpu_sc as plsc

PRE_SEQ_LEN = 128
ROW_DIM = 98304
BATCH = 4
B_FLAT = 512

NUM_CORES = 2
NUM_SUBCORES = 16
ROWS_PER_TEC = B_FLAT // NUM_SUBCORES
A_ROWS = 16
B_ROWS = ROWS_PER_TEC - A_ROWS
HALF_COLS = ROW_DIM // NUM_CORES
CHUNK = 2048
NUM_CHUNKS = HALF_COLS // CHUNK
SUB = 1024
NUM_SUBS = CHUNK // SUB
STAGE_ROWS = PRE_SEQ_LEN // NUM_SUBCORES
UNROLL = 6


def make_sc_gather():
    mesh = plsc.VectorSubcoreMesh(core_axis_name="c", subcore_axis_name="s")

    @functools.partial(
        pl.kernel,
        out_type=jax.ShapeDtypeStruct((B_FLAT, ROW_DIM), jnp.float32),
        mesh=mesh,
        scratch_types=[
            pltpu.VMEM((ROWS_PER_TEC,), jnp.int32),
            pltpu.VMEM((B_ROWS, SUB), jnp.float32),
            pltpu.VMEM((B_ROWS, SUB), jnp.float32),
            pltpu.VMEM_SHARED((PRE_SEQ_LEN, CHUNK), jnp.float32),
            pltpu.VMEM_SHARED((PRE_SEQ_LEN, CHUNK), jnp.float32),
            pltpu.VMEM_SHARED((PRE_SEQ_LEN, CHUNK), jnp.float32),
            pltpu.SemaphoreType.DMA,
            pltpu.SemaphoreType.DMA,
            pltpu.SemaphoreType.DMA,
            pltpu.SemaphoreType.DMA,
            pltpu.SemaphoreType.DMA,
            pltpu.SemaphoreType.DMA,
            pltpu.SemaphoreType.DMA,
        ],
    )
    def sc_gather(
        idx_hbm, table_hbm, out_hbm,
        idx_v, tb0, tb1, sp0, sp1, sp2,
        st_sem, xb_sem, aw0, aw1, aw2, bw0, bw1,
    ):
        sc = lax.axis_index("c")
        tec = lax.axis_index("s")
        row0 = tec * ROWS_PER_TEC
        col0 = sc * HALF_COLS
        srow = tec * STAGE_ROWS
        pltpu.sync_copy(idx_hbm.at[pl.ds(row0, ROWS_PER_TEC)], idx_v)
        idx_lo = idx_v[pl.ds(0, 16)]
        idx_hi = idx_v[pl.ds(16, 16)]
        idx_s = [idx_lo[i] for i in range(16)] + [idx_hi[i] for i in range(16)]

        sps = (sp0, sp1, sp2)
        aws = (aw0, aw1, aw2)
        tbs = (tb0, tb1)
        bws = (bw0, bw1)

        def stage(c, k3):
            return pltpu.make_async_copy(
                table_hbm.at[pl.ds(srow, STAGE_ROWS), pl.ds(col0 + c * CHUNK, CHUNK)],
                sps[k3].at[pl.ds(srow, STAGE_ROWS)],
                st_sem,
            )

        def a_write(c, k3, i):
            return pltpu.make_async_copy(
                sps[k3].at[pl.ds(idx_s[i], 1)],
                out_hbm.at[pl.ds(row0 + i, 1), pl.ds(col0 + c * CHUNK, CHUNK)],
                aws[k3],
            )

        def b_fill(k3, s, q2, i):
            return pltpu.make_async_copy(
                sps[k3].at[pl.ds(idx_s[A_ROWS + i], 1), pl.ds(s * SUB, SUB)],
                tbs[q2].at[pl.ds(i, 1)],
                xb_sem,
            )

        def b_write(c, s, q2):
            return pltpu.make_async_copy(
                tbs[q2],
                out_hbm.at[
                    pl.ds(row0 + A_ROWS, B_ROWS),
                    pl.ds(col0 + c * CHUNK + s * SUB, SUB),
                ],
                bws[q2],
            )

        def step(c, u):
            k3 = u % 3
            stage(c, k3).wait()

            @pl.when(c >= 2)
            def _drain_a():
                for i in range(A_ROWS):
                    a_write(c - 2, (u - 2) % 3, i).wait()

            plsc.subcore_barrier()

            @pl.when(c + 1 < NUM_CHUNKS)
            def _next_stage():
                stage(c + 1, (u + 1) % 3).start()

            for i in range(A_ROWS):
                a_write(c, k3, i).start()

            for s in range(NUM_SUBS):
                t = u * NUM_SUBS + s
                q2 = t % 2

                @pl.when(c * NUM_SUBS + s >= 2)
                def _drain_b(c=c, s=s, q2=q2):
                    cc = c if s >= 2 else c - 1
                    ss = s - 2 if s >= 2 else s + NUM_SUBS - 2
                    b_write(cc, ss, q2).wait()

                for i in range(B_ROWS):
                    b_fill(k3, s, q2, i).start()
                for i in range(B_ROWS):
                    b_fill(k3, s, q2, i).wait()
                b_write(c, s, q2).start()

        stage(0, 0).start()

        def body(j, _):
            for u in range(UNROLL):
                step(UNROLL * j + u, u)
            return _

        lax.fori_loop(0, NUM_CHUNKS // UNROLL, body, 0, unroll=False)
        for i in range(A_ROWS):
            a_write(NUM_CHUNKS - 2, (NUM_CHUNKS - 2) % 3, i).wait()
        for i in range(A_ROWS):
            a_write(NUM_CHUNKS - 1, (NUM_CHUNKS - 1) % 3, i).wait()
        b_write(NUM_CHUNKS - 1, NUM_SUBS - 2, 0).wait()
        b_write(NUM_CHUNKS - 1, NUM_SUBS - 1, 1).wait()

    return sc_gather


def kernel(prefix, embedding):
    idx = prefix.reshape(B_FLAT)
    out = make_sc_gather()(idx, embedding)
    return out.reshape(BATCH, PRE_SEQ_LEN, ROW_DIM)


# speedup vs baseline: 2.5301x; 1.0093x over previous
import functools

import jax
import jax.numpy as jnp
from jax import lax
from jax.experimental import pallas as pl
from jax.experimental.pallas import tpu as pltpu
from jax.experimental.pallas import tpu_sc as plsc

PRE_SEQ_LEN = 128
ROW_DIM = 98304
BATCH = 4
B_FLAT = 512

NUM_CORES = 2
NUM_SUBCORES = 16
ROWS_PER_TEC = B_FLAT // NUM_SUBCORES
A_ROWS = 16
B_ROWS = ROWS_PER_TEC - A_ROWS
HALF_COLS = ROW_DIM // NUM_CORES
CHUNK = 2048
NUM_CHUNKS = HALF_COLS // CHUNK
SUB = 2048
NUM_SUBS = CHUNK // SUB
STAGE_ROWS = PRE_SEQ_LEN // NUM_SUBCORES
UNROLL = 6


def make_sc_gather():
    mesh = plsc.VectorSubcoreMesh(core_axis_name="c", subcore_axis_name="s")

    @functools.partial(
        pl.kernel,
        out_type=jax.ShapeDtypeStruct((B_FLAT, ROW_DIM), jnp.float32),
        mesh=mesh,
        scratch_types=[
            pltpu.VMEM((ROWS_PER_TEC,), jnp.int32),
            pltpu.VMEM((B_ROWS, SUB), jnp.float32),
            pltpu.VMEM((B_ROWS, SUB), jnp.float32),
            pltpu.VMEM_SHARED((PRE_SEQ_LEN, CHUNK), jnp.float32),
            pltpu.VMEM_SHARED((PRE_SEQ_LEN, CHUNK), jnp.float32),
            pltpu.VMEM_SHARED((PRE_SEQ_LEN, CHUNK), jnp.float32),
            pltpu.SemaphoreType.DMA,
            pltpu.SemaphoreType.DMA,
            pltpu.SemaphoreType.DMA,
            pltpu.SemaphoreType.DMA,
            pltpu.SemaphoreType.DMA,
            pltpu.SemaphoreType.DMA,
            pltpu.SemaphoreType.DMA,
        ],
    )
    def sc_gather(
        idx_hbm, table_hbm, out_hbm,
        idx_v, tb0, tb1, sp0, sp1, sp2,
        st_sem, xb_sem, aw0, aw1, aw2, bw0, bw1,
    ):
        sc = lax.axis_index("c")
        tec = lax.axis_index("s")
        row0 = tec * ROWS_PER_TEC
        col0 = sc * HALF_COLS
        srow = tec * STAGE_ROWS
        pltpu.sync_copy(idx_hbm.at[pl.ds(row0, ROWS_PER_TEC)], idx_v)
        idx_lo = idx_v[pl.ds(0, 16)]
        idx_hi = idx_v[pl.ds(16, 16)]
        idx_s = [idx_lo[i] for i in range(16)] + [idx_hi[i] for i in range(16)]

        sps = (sp0, sp1, sp2)
        aws = (aw0, aw1, aw2)
        tbs = (tb0, tb1)
        bws = (bw0, bw1)

        def stage(c, k3):
            return pltpu.make_async_copy(
                table_hbm.at[pl.ds(srow, STAGE_ROWS), pl.ds(col0 + c * CHUNK, CHUNK)],
                sps[k3].at[pl.ds(srow, STAGE_ROWS)],
                st_sem,
            )

        def a_write(c, k3, i):
            return pltpu.make_async_copy(
                sps[k3].at[pl.ds(idx_s[i], 1)],
                out_hbm.at[pl.ds(row0 + i, 1), pl.ds(col0 + c * CHUNK, CHUNK)],
                aws[k3],
            )

        def b_fill(k3, s, q2, i):
            return pltpu.make_async_copy(
                sps[k3].at[pl.ds(idx_s[A_ROWS + i], 1), pl.ds(s * SUB, SUB)],
                tbs[q2].at[pl.ds(i, 1)],
                xb_sem,
            )

        def b_write(c, s, q2):
            return pltpu.make_async_copy(
                tbs[q2],
                out_hbm.at[
                    pl.ds(row0 + A_ROWS, B_ROWS),
                    pl.ds(col0 + c * CHUNK + s * SUB, SUB),
                ],
                bws[q2],
            )

        def step(c, u):
            k3 = u % 3
            stage(c, k3).wait()

            @pl.when(c >= 2)
            def _drain_a():
                for i in range(A_ROWS):
                    a_write(c - 2, (u - 2) % 3, i).wait()

            plsc.subcore_barrier()

            @pl.when(c + 1 < NUM_CHUNKS)
            def _next_stage():
                stage(c + 1, (u + 1) % 3).start()

            for i in range(A_ROWS):
                a_write(c, k3, i).start()

            for s in range(NUM_SUBS):
                t = u * NUM_SUBS + s
                q2 = t % 2

                ss = (s - 2) % NUM_SUBS
                borrow = 0 if s >= 2 else (2 - s + NUM_SUBS - 1) // NUM_SUBS

                @pl.when(c * NUM_SUBS + s >= 2)
                def _drain_b(c=c, ss=ss, borrow=borrow, q2=q2):
                    b_write(c - borrow, ss, q2).wait()

                for i in range(B_ROWS):
                    b_fill(k3, s, q2, i).start()
                for i in range(B_ROWS):
                    b_fill(k3, s, q2, i).wait()
                b_write(c, s, q2).start()

        stage(0, 0).start()

        def body(j, _):
            for u in range(UNROLL):
                step(UNROLL * j + u, u)
            return _

        lax.fori_loop(0, NUM_CHUNKS // UNROLL, body, 0, unroll=False)
        for i in range(A_ROWS):
            a_write(NUM_CHUNKS - 2, (NUM_CHUNKS - 2) % 3, i).wait()
        for i in range(A_ROWS):
            a_write(NUM_CHUNKS - 1, (NUM_CHUNKS - 1) % 3, i).wait()
        total_subs = NUM_CHUNKS * NUM_SUBS
        for g in (total_subs - 2, total_subs - 1):
            cg, sg = g // NUM_SUBS, g % NUM_SUBS
            qg = ((cg % UNROLL) * NUM_SUBS + sg) % 2
            b_write(cg, sg, qg).wait()

    return sc_gather


def kernel(prefix, embedding):
    idx = prefix.reshape(B_FLAT)
    out = make_sc_gather()(idx, embedding)
    return out.reshape(BATCH, PRE_SEQ_LEN, ROW_DIM)
